# baseline (speedup 1.0000x reference)
import functools

import numpy as np
import jax
import jax.numpy as jnp
from jax import lax
from jax.experimental import pallas as pl
from jax.experimental.pallas import tpu as pltpu
from jax.experimental.pallas import tpu_sc as plsc

_N = 512
_B = 128
_NCHUNK = 16
_NW = 32
_BPW = _B // _NW

_tri = np.triu(np.ones((_N, _N), dtype=bool), k=0)
_rows_np, _cols_np = np.nonzero(_tri)
_M = _rows_np.size
_P = _M // _NCHUNK
_VPC = _P // 16

_RLO = []
_SPAN = []
_tbl_np = np.empty((_NCHUNK, _P), dtype=np.int32)
for _c in range(_NCHUNK):
    _sl = slice(_c * _P, (_c + 1) * _P)
    _rlo = int(_rows_np[_c * _P])
    _rhi = int(_rows_np[(_c + 1) * _P - 1])
    _RLO.append(_rlo)
    _SPAN.append(_rhi - _rlo + 1)
    _tbl_np[_c] = ((_rows_np[_sl] - _rlo) * _N + _cols_np[_sl]).astype(np.int32)

_IN_BUF = max(_SPAN) * _N


@functools.partial(
    pl.kernel,
    mesh=plsc.VectorSubcoreMesh(core_axis_name="c", subcore_axis_name="s"),
    compiler_params=pltpu.CompilerParams(needs_layout_passes=False),
    out_type=jax.ShapeDtypeStruct((_B * _M,), jnp.float32),
    scratch_types=[
        pltpu.VMEM((_IN_BUF,), jnp.float32),
        pltpu.VMEM((_P,), jnp.int32),
        pltpu.VMEM((_P,), jnp.float32),
    ],
)
def _triu_select(in_hbm, tbl_hbm, out_hbm, in_buf, tbl_buf, out_buf):
    wid = lax.axis_index("s") * 2 + lax.axis_index("c")

    for c in range(_NCHUNK):
        pltpu.sync_copy(tbl_hbm.at[pl.ds(c * _P, _P)], tbl_buf)
        span = _SPAN[c] * _N
        base = _RLO[c] * _N

        def batch_body(i, _, c=c, span=span, base=base):
            b = wid * _BPW + i
            pltpu.sync_copy(in_hbm.at[pl.ds(b * (_N * _N) + base, span)],
                            in_buf.at[pl.ds(0, span)])

            def vec_body(j, _):
                idx = tbl_buf[pl.ds(j * 16, 16)]
                out_buf[pl.ds(j * 16, 16)] = plsc.load_gather(in_buf, [idx])
                return 0

            lax.fori_loop(0, _VPC, vec_body, 0)
            pltpu.sync_copy(out_buf, out_hbm.at[pl.ds(b * _M + c * _P, _P)])
            return 0

        lax.fori_loop(0, _BPW, batch_body, 0)


def kernel(input):
    tbl = jnp.asarray(_tbl_np.reshape(-1))
    out = _triu_select(input.reshape(_B * _N * _N), tbl)
    return out.reshape(_B, _M)


# speedup vs baseline: 1.4270x; 1.4270x over previous
import functools

import numpy as np
import jax
import jax.numpy as jnp
from jax import lax
from jax.experimental import pallas as pl
from jax.experimental.pallas import tpu as pltpu
from jax.experimental.pallas import tpu_sc as plsc

_N = 512
_B = 128
_NCHUNK = 16
_NW = 32
_BPW = _B // _NW

_tri = np.triu(np.ones((_N, _N), dtype=bool), k=0)
_rows_np, _cols_np = np.nonzero(_tri)
_M = _rows_np.size
_P = _M // _NCHUNK
_VPC = _P // 16

_RLO = []
_SPAN = []
_tbl_np = np.empty((_NCHUNK, _P), dtype=np.int32)
for _c in range(_NCHUNK):
    _sl = slice(_c * _P, (_c + 1) * _P)
    _rlo = int(_rows_np[_c * _P])
    _rhi = int(_rows_np[(_c + 1) * _P - 1])
    _RLO.append(_rlo)
    _SPAN.append(_rhi - _rlo + 1)
    _tbl_np[_c] = ((_rows_np[_sl] - _rlo) * _N + _cols_np[_sl]).astype(np.int32)

_IN_BUF = max(_SPAN) * _N


@functools.partial(
    pl.kernel,
    mesh=plsc.VectorSubcoreMesh(core_axis_name="c", subcore_axis_name="s"),
    compiler_params=pltpu.CompilerParams(needs_layout_passes=False),
    out_type=jax.ShapeDtypeStruct((_B * _M,), jnp.float32),
    scratch_types=[
        pltpu.VMEM((_IN_BUF,), jnp.float32),
        pltpu.VMEM((_P,), jnp.int32),
        pltpu.VMEM((_P,), jnp.float32),
    ],
)
def _triu_select(in_hbm, tbl_hbm, out_hbm, in_buf, tbl_buf, out_buf):
    wid = lax.axis_index("s") * 2 + lax.axis_index("c")

    for c in range(_NCHUNK):
        pltpu.sync_copy(tbl_hbm.at[pl.ds(c * _P, _P)], tbl_buf)
        span = _SPAN[c] * _N
        base = _RLO[c] * _N

        def batch_body(i, _, c=c, span=span, base=base):
            b = wid * _BPW + i
            pltpu.sync_copy(in_hbm.at[pl.ds(b * (_N * _N) + base, span)],
                            in_buf.at[pl.ds(0, span)])

            @plsc.parallel_loop(0, _P, 16, unroll=8)
            def vec_body(j):
                idx = tbl_buf[pl.ds(j, 16)]
                out_buf[pl.ds(j, 16)] = plsc.load_gather(in_buf, [idx])
            pltpu.sync_copy(out_buf, out_hbm.at[pl.ds(b * _M + c * _P, _P)])
            return 0

        lax.fori_loop(0, _BPW, batch_body, 0)


def kernel(input):
    tbl = jnp.asarray(_tbl_np.reshape(-1))
    out = _triu_select(input.reshape(_B * _N * _N), tbl)
    return out.reshape(_B, _M)


# speedup vs baseline: 1.7584x; 1.2323x over previous
import functools

import numpy as np
import jax
import jax.numpy as jnp
from jax import lax
from jax.experimental import pallas as pl
from jax.experimental.pallas import tpu as pltpu
from jax.experimental.pallas import tpu_sc as plsc

_N = 512
_B = 128
_NCHUNK = 16
_ROWS_PER_CHUNK = _N // _NCHUNK
_IN_WIN = _ROWS_PER_CHUNK * _N
_NW = 32
_BPW = _B // _NW

_tri = np.triu(np.ones((_N, _N), dtype=bool), k=0)
_rows_np, _cols_np = np.nonzero(_tri)
_M = _rows_np.size

_O = []
_S = []
for _c in range(_NCHUNK):
    _r0 = _c * _ROWS_PER_CHUNK
    _off = _r0 * _N - _r0 * (_r0 - 1) // 2
    _O.append(_off)
_O.append(_M)
for _c in range(_NCHUNK):
    _S.append(_O[_c + 1] - _O[_c])
_SMAX = max(_S)

_tbl_np = np.empty((_M,), dtype=np.int32)
for _c in range(_NCHUNK):
    _sl = slice(_O[_c], _O[_c + 1])
    _tbl_np[_sl] = ((_rows_np[_sl] - _c * _ROWS_PER_CHUNK) * _N
                    + _cols_np[_sl]).astype(np.int32)


@functools.partial(
    pl.kernel,
    mesh=plsc.VectorSubcoreMesh(core_axis_name="c", subcore_axis_name="s"),
    compiler_params=pltpu.CompilerParams(needs_layout_passes=False),
    out_type=jax.ShapeDtypeStruct((_B * _M,), jnp.float32),
    scratch_types=[
        pltpu.VMEM((_IN_WIN,), jnp.float32),
        pltpu.VMEM((_IN_WIN,), jnp.float32),
        pltpu.VMEM((_SMAX,), jnp.int32),
        pltpu.VMEM((_SMAX,), jnp.int32),
        pltpu.VMEM((_SMAX,), jnp.float32),
        pltpu.VMEM((_SMAX,), jnp.float32),
        pltpu.SemaphoreType.DMA,
        pltpu.SemaphoreType.DMA,
        pltpu.SemaphoreType.DMA,
        pltpu.SemaphoreType.DMA,
        pltpu.SemaphoreType.DMA,
        pltpu.SemaphoreType.DMA,
    ],
)
def _triu_select(in_hbm, tbl_hbm, out_hbm,
                 in0, in1, tb0, tb1, ob0, ob1,
                 isem0, isem1, tsem0, tsem1, osem0, osem1):
    wid = lax.axis_index("s") * 2 + lax.axis_index("c")
    in_bufs, tbl_bufs, out_bufs = (in0, in1), (tb0, tb1), (ob0, ob1)
    in_sems, tbl_sems, out_sems = (isem0, isem1), (tsem0, tsem1), (osem0, osem1)

    units = [(c, i) for c in range(_NCHUNK) for i in range(_BPW)]

    def in_copy(u):
        c, i = units[u]
        b = wid * _BPW + i
        return pltpu.make_async_copy(
            in_hbm.at[pl.ds(b * (_N * _N) + c * _IN_WIN, _IN_WIN)],
            in_bufs[u % 2], in_sems[u % 2])

    def tbl_copy(c):
        return pltpu.make_async_copy(
            tbl_hbm.at[pl.ds(_O[c], _S[c])],
            tbl_bufs[c % 2].at[pl.ds(0, _S[c])], tbl_sems[c % 2])

    def out_copy(u):
        c, i = units[u]
        b = wid * _BPW + i
        return pltpu.make_async_copy(
            out_bufs[u % 2].at[pl.ds(0, _S[c])],
            out_hbm.at[pl.ds(b * _M + _O[c], _S[c])], out_sems[u % 2])

    tbl_copy(0).start()
    in_copy(0).start()
    for u, (c, i) in enumerate(units):
        if i == 0 and c + 1 < _NCHUNK:
            tbl_copy(c + 1).start()
        if u + 1 < len(units):
            in_copy(u + 1).start()
        in_copy(u).wait()
        if i == 0:
            tbl_copy(c).wait()
        if u >= 2:
            out_copy(u - 2).wait()

        in_buf, tbl_buf, out_buf = in_bufs[u % 2], tbl_bufs[c % 2], out_bufs[u % 2]

        @plsc.parallel_loop(0, _S[c], 16, unroll=8)
        def vec_body(j):
            idx = tbl_buf[pl.ds(j, 16)]
            out_buf[pl.ds(j, 16)] = plsc.load_gather(in_buf, [idx])

        out_copy(u).start()
    out_copy(len(units) - 2).wait()
    out_copy(len(units) - 1).wait()


def kernel(input):
    tbl = jnp.asarray(_tbl_np)
    out = _triu_select(input.reshape(_B * _N * _N), tbl)
    return out.reshape(_B, _M)


# speedup vs baseline: 2.4759x; 1.4080x over previous
import functools

import numpy as np
import jax
import jax.numpy as jnp
from jax import lax
from jax.experimental import pallas as pl
from jax.experimental.pallas import tpu as pltpu
from jax.experimental.pallas import tpu_sc as plsc

_N = 512
_B = 128
_NCHUNK = 16
_ROWS_PER_CHUNK = _N // _NCHUNK
_IN_WIN = _ROWS_PER_CHUNK * _N
_NW = 32
_BPW = _B // _NW

_tri = np.triu(np.ones((_N, _N), dtype=bool), k=0)
_rows_np, _cols_np = np.nonzero(_tri)
_M = _rows_np.size

_O = []
_S = []
for _c in range(_NCHUNK):
    _r0 = _c * _ROWS_PER_CHUNK
    _off = _r0 * _N - _r0 * (_r0 - 1) // 2
    _O.append(_off)
_O.append(_M)
for _c in range(_NCHUNK):
    _S.append(_O[_c + 1] - _O[_c])
_SMAX = max(_S)

_tbl_np = np.empty((_M,), dtype=np.int32)
for _c in range(_NCHUNK):
    _sl = slice(_O[_c], _O[_c + 1])
    _tbl_np[_sl] = ((_rows_np[_sl] - _c * _ROWS_PER_CHUNK) * _N
                    + _cols_np[_sl]).astype(np.int32)


@functools.partial(
    pl.kernel,
    mesh=plsc.VectorSubcoreMesh(core_axis_name="c", subcore_axis_name="s"),
    compiler_params=pltpu.CompilerParams(needs_layout_passes=False),
    out_type=jax.ShapeDtypeStruct((_B * _M,), jnp.float32),
    scratch_types=[
        pltpu.VMEM((_ROWS_PER_CHUNK, _N), jnp.float32),
        pltpu.VMEM((_ROWS_PER_CHUNK, _N), jnp.float32),
        pltpu.VMEM((_SMAX,), jnp.int32),
        pltpu.VMEM((_SMAX,), jnp.int32),
        pltpu.VMEM((_SMAX,), jnp.float32),
        pltpu.VMEM((_SMAX,), jnp.float32),
        pltpu.SemaphoreType.DMA,
        pltpu.SemaphoreType.DMA,
        pltpu.SemaphoreType.DMA,
        pltpu.SemaphoreType.DMA,
        pltpu.SemaphoreType.DMA,
        pltpu.SemaphoreType.DMA,
    ],
)
def _triu_select(in_hbm, tbl_hbm, out_hbm,
                 in0, in1, tb0, tb1, ob0, ob1,
                 isem0, isem1, tsem0, tsem1, osem0, osem1):
    wid = lax.axis_index("s") * 2 + lax.axis_index("c")
    in_bufs, tbl_bufs, out_bufs = (in0, in1), (tb0, tb1), (ob0, ob1)
    in_sems, tbl_sems, out_sems = (isem0, isem1), (tsem0, tsem1), (osem0, osem1)

    units = [(c, i) for c in range(_NCHUNK) for i in range(_BPW)]

    def in_copy(u):
        c, i = units[u]
        b = wid * _BPW + i
        return pltpu.make_async_copy(
            in_hbm.at[b, pl.ds(c * _ROWS_PER_CHUNK, _ROWS_PER_CHUNK), :],
            in_bufs[u % 2], in_sems[u % 2])

    def tbl_copy(c):
        return pltpu.make_async_copy(
            tbl_hbm.at[pl.ds(_O[c], _S[c])],
            tbl_bufs[c % 2].at[pl.ds(0, _S[c])], tbl_sems[c % 2])

    def out_copy(u):
        c, i = units[u]
        b = wid * _BPW + i
        return pltpu.make_async_copy(
            out_bufs[u % 2].at[pl.ds(0, _S[c])],
            out_hbm.at[pl.ds(b * _M + _O[c], _S[c])], out_sems[u % 2])

    tbl_copy(0).start()
    in_copy(0).start()
    for u, (c, i) in enumerate(units):
        if i == 0 and c + 1 < _NCHUNK:
            tbl_copy(c + 1).start()
        if u + 1 < len(units):
            in_copy(u + 1).start()
        in_copy(u).wait()
        if i == 0:
            tbl_copy(c).wait()
        if u >= 2:
            out_copy(u - 2).wait()

        in_buf, tbl_buf, out_buf = in_bufs[u % 2], tbl_bufs[c % 2], out_bufs[u % 2]

        @plsc.parallel_loop(0, _S[c], 16, unroll=8)
        def vec_body(j):
            idx = tbl_buf[pl.ds(j, 16)]
            out_buf[pl.ds(j, 16)] = plsc.load_gather(
                in_buf, [lax.shift_right_logical(idx, 9),
                         lax.bitwise_and(idx, _N - 1)])

        out_copy(u).start()
    out_copy(len(units) - 2).wait()
    out_copy(len(units) - 1).wait()


def kernel(input):
    tbl = jnp.asarray(_tbl_np)
    out = _triu_select(input, tbl)
    return out.reshape(_B, _M)
